# Initial kernel scaffold; baseline (speedup 1.0000x reference)
import jax
import jax.numpy as jnp
from jax.experimental import pallas as pl


def kernel(graph_feat, W1, b1, W2, b2, W3, b3):
    raise NotImplementedError("write your pallas kernel here")


# speedup vs baseline: 10.1091x; 10.1091x over previous
import jax
import jax.numpy as jnp
from jax.experimental import pallas as pl

N = 4096
D_IN = 256
D_HID = 512
TOP_K = 10

_MLP_BLK = 512
_ROW_BLK = 128


def _mlp_norm_kernel(x_ref, w1_ref, b1_ref, w2_ref, b2_ref, w3_ref, b3_ref,
                     out_ref):
    x = x_ref[...]
    h = jax.lax.dot_general(x, w1_ref[...], (((1,), (1,)), ((), ())),
                            preferred_element_type=jnp.float32)
    h = jnp.maximum(h + b1_ref[...], 0.0)
    h = jax.lax.dot_general(h, w2_ref[...], (((1,), (1,)), ((), ())),
                            preferred_element_type=jnp.float32)
    h = jnp.maximum(h + b2_ref[...], 0.0)
    h = jax.lax.dot_general(h, w3_ref[...], (((1,), (1,)), ((), ())),
                            preferred_element_type=jnp.float32)
    feat = x + h + b3_ref[...]
    nrm = jnp.sqrt(jnp.sum(feat * feat, axis=1, keepdims=True)) + 1e-8
    out_ref[...] = feat / nrm


def _thresh_kernel(nb_ref, nf_ref, th_ref):
    sim = jax.lax.dot_general(nb_ref[...], nf_ref[...],
                              (((1,), (1,)), ((), ())),
                              preferred_element_type=jnp.float32)
    col = jax.lax.broadcasted_iota(jnp.int32, sim.shape, 1)
    cur = sim
    for _ in range(TOP_K - 1):
        m = jnp.max(cur, axis=1, keepdims=True)
        hit = jnp.where(cur == m, col, N)
        first = jnp.min(hit, axis=1, keepdims=True)
        cur = jnp.where(col == first, -jnp.inf, cur)
    th_ref[...] = jnp.max(cur, axis=1, keepdims=True)


def _mask_kernel(nb_ref, nf_ref, thr_ref, thc_ref, out_ref):
    sim = jax.lax.dot_general(nb_ref[...], nf_ref[...],
                              (((1,), (1,)), ((), ())),
                              preferred_element_type=jnp.float32)
    a = jnp.where(sim >= thr_ref[...], sim, 0.0)
    b = jnp.where(sim >= thc_ref[...], sim, 0.0)
    out_ref[...] = jnp.maximum(a, b)


def kernel(graph_feat, W1, b1, W2, b2, W3, b3):
    b1r = b1.reshape(1, D_HID)
    b2r = b2.reshape(1, D_HID)
    b3r = b3.reshape(1, D_IN)

    whole = lambda shape: pl.BlockSpec(shape, lambda i: (0, 0))

    normed = pl.pallas_call(
        _mlp_norm_kernel,
        grid=(N // _MLP_BLK,),
        in_specs=[
            pl.BlockSpec((_MLP_BLK, D_IN), lambda i: (i, 0)),
            whole((D_HID, D_IN)),
            whole((1, D_HID)),
            whole((D_HID, D_HID)),
            whole((1, D_HID)),
            whole((D_IN, D_HID)),
            whole((1, D_IN)),
        ],
        out_specs=pl.BlockSpec((_MLP_BLK, D_IN), lambda i: (i, 0)),
        out_shape=jax.ShapeDtypeStruct((N, D_IN), jnp.float32),
    )(graph_feat, W1, b1r, W2, b2r, W3, b3r)

    th = pl.pallas_call(
        _thresh_kernel,
        grid=(N // _ROW_BLK,),
        in_specs=[
            pl.BlockSpec((_ROW_BLK, D_IN), lambda i: (i, 0)),
            whole((N, D_IN)),
        ],
        out_specs=pl.BlockSpec((_ROW_BLK, 1), lambda i: (i, 0)),
        out_shape=jax.ShapeDtypeStruct((N, 1), jnp.float32),
    )(normed, normed)

    adj = pl.pallas_call(
        _mask_kernel,
        grid=(N // _ROW_BLK,),
        in_specs=[
            pl.BlockSpec((_ROW_BLK, D_IN), lambda i: (i, 0)),
            whole((N, D_IN)),
            pl.BlockSpec((_ROW_BLK, 1), lambda i: (i, 0)),
            whole((1, N)),
        ],
        out_specs=pl.BlockSpec((_ROW_BLK, N), lambda i: (i, 0)),
        out_shape=jax.ShapeDtypeStruct((N, N), jnp.float32),
    )(normed, normed, th, th.reshape(1, N))

    return adj


# speedup vs baseline: 22.3756x; 2.2134x over previous
import jax
import jax.numpy as jnp
from jax.experimental import pallas as pl

N = 4096
D_IN = 256
D_HID = 512
TOP_K = 10

_MLP_BLK = 512
_ROW_BLK = 128


def _mlp_norm_kernel(x_ref, w1_ref, b1_ref, w2_ref, b2_ref, w3_ref, b3_ref,
                     out_ref):
    x = x_ref[...]
    h = jax.lax.dot_general(x, w1_ref[...], (((1,), (1,)), ((), ())),
                            preferred_element_type=jnp.float32)
    h = jnp.maximum(h + b1_ref[...], 0.0)
    h = jax.lax.dot_general(h, w2_ref[...], (((1,), (1,)), ((), ())),
                            preferred_element_type=jnp.float32)
    h = jnp.maximum(h + b2_ref[...], 0.0)
    h = jax.lax.dot_general(h, w3_ref[...], (((1,), (1,)), ((), ())),
                            preferred_element_type=jnp.float32)
    feat = x + h + b3_ref[...]
    nrm = jnp.sqrt(jnp.sum(feat * feat, axis=1, keepdims=True)) + 1e-8
    out_ref[...] = feat / nrm


def _merge22(A, B):
    e0 = jnp.maximum(A[0], B[1])
    e2 = jnp.minimum(A[0], B[1])
    e1 = jnp.maximum(A[1], B[0])
    e3 = jnp.minimum(A[1], B[0])
    return (jnp.maximum(e0, e1), jnp.minimum(e0, e1),
            jnp.maximum(e2, e3), jnp.minimum(e2, e3))


def _merge44_top4(A, B):
    t0 = jnp.maximum(A[0], B[3])
    t1 = jnp.maximum(A[1], B[2])
    t2 = jnp.maximum(A[2], B[1])
    t3 = jnp.maximum(A[3], B[0])
    u0 = jnp.maximum(t0, t2)
    u2 = jnp.minimum(t0, t2)
    u1 = jnp.maximum(t1, t3)
    u3 = jnp.minimum(t1, t3)
    return (jnp.maximum(u0, u1), jnp.minimum(u0, u1),
            jnp.maximum(u2, u3), jnp.minimum(u2, u3))


def _thresh_kernel(nb_ref, nf_ref, th_ref):
    sim = jax.lax.dot_general(nb_ref[...], nf_ref[...],
                              (((1,), (1,)), ((), ())),
                              preferred_element_type=jnp.float32)
    planes = [sim[:, 128 * g:128 * (g + 1)] for g in range(32)]
    l2 = [(jnp.maximum(planes[2 * i], planes[2 * i + 1]),
           jnp.minimum(planes[2 * i], planes[2 * i + 1])) for i in range(16)]
    l4 = [_merge22(l2[2 * i], l2[2 * i + 1]) for i in range(8)]
    while len(l4) > 1:
        l4 = [_merge44_top4(l4[2 * i], l4[2 * i + 1])
              for i in range(len(l4) // 2)]
    cand = jnp.concatenate(l4[0], axis=1)
    cur = cand
    for _ in range(TOP_K - 1):
        m = jnp.max(cur, axis=1, keepdims=True)
        cur = jnp.where(cur == m, -jnp.inf, cur)
    th_ref[...] = jnp.max(cur, axis=1, keepdims=True)


def _mask_kernel(nb_ref, nf_ref, thr_ref, thc_ref, out_ref):
    sim = jax.lax.dot_general(nb_ref[...], nf_ref[...],
                              (((1,), (1,)), ((), ())),
                              preferred_element_type=jnp.float32)
    a = jnp.where(sim >= thr_ref[...], sim, 0.0)
    b = jnp.where(sim >= thc_ref[...], sim, 0.0)
    out_ref[...] = jnp.maximum(a, b)


def kernel(graph_feat, W1, b1, W2, b2, W3, b3):
    b1r = b1.reshape(1, D_HID)
    b2r = b2.reshape(1, D_HID)
    b3r = b3.reshape(1, D_IN)

    whole = lambda shape: pl.BlockSpec(shape, lambda i: (0, 0))

    normed = pl.pallas_call(
        _mlp_norm_kernel,
        grid=(N // _MLP_BLK,),
        in_specs=[
            pl.BlockSpec((_MLP_BLK, D_IN), lambda i: (i, 0)),
            whole((D_HID, D_IN)),
            whole((1, D_HID)),
            whole((D_HID, D_HID)),
            whole((1, D_HID)),
            whole((D_IN, D_HID)),
            whole((1, D_IN)),
        ],
        out_specs=pl.BlockSpec((_MLP_BLK, D_IN), lambda i: (i, 0)),
        out_shape=jax.ShapeDtypeStruct((N, D_IN), jnp.float32),
    )(graph_feat, W1, b1r, W2, b2r, W3, b3r)

    th = pl.pallas_call(
        _thresh_kernel,
        grid=(N // _ROW_BLK,),
        in_specs=[
            pl.BlockSpec((_ROW_BLK, D_IN), lambda i: (i, 0)),
            whole((N, D_IN)),
        ],
        out_specs=pl.BlockSpec((_ROW_BLK, 1), lambda i: (i, 0)),
        out_shape=jax.ShapeDtypeStruct((N, 1), jnp.float32),
    )(normed, normed)

    adj = pl.pallas_call(
        _mask_kernel,
        grid=(N // _ROW_BLK,),
        in_specs=[
            pl.BlockSpec((_ROW_BLK, D_IN), lambda i: (i, 0)),
            whole((N, D_IN)),
            pl.BlockSpec((_ROW_BLK, 1), lambda i: (i, 0)),
            whole((1, N)),
        ],
        out_specs=pl.BlockSpec((_ROW_BLK, N), lambda i: (i, 0)),
        out_shape=jax.ShapeDtypeStruct((N, N), jnp.float32),
    )(normed, normed, th, th.reshape(1, N))

    return adj
